# baseline (speedup 1.0000x reference)
import functools

import jax
import jax.numpy as jnp
from jax import lax
from jax.experimental import pallas as pl
from jax.experimental.pallas import tpu as pltpu
from jax.experimental.pallas import tpu_sc as plsc

N = 10000
NP = 10240
E = 320000
NC = 2
NS = 16
NW = NC * NS
EPW = E // NW
CH = 80
NCH = EPW // CH
CH2 = 40
NCH2 = EPW // CH2
CH64 = 128
NCH64 = 79
EP = NW * CH64 * NCH64
EPAD = EP - E
RPT = NP // NS
RPT2 = N - (NS - 1) * RPT
DEG_W = 8

_mesh = plsc.VectorSubcoreMesh(core_axis_name="c", subcore_axis_name="s")


def _make_prop(D, ch, nch, nbuf):
  @functools.partial(
      pl.kernel,
      out_type=jax.ShapeDtypeStruct((NC, NP, D), jnp.float32),
      mesh=_mesh,
      compiler_params=pltpu.CompilerParams(use_tc_tiling_on_sc=False),
      scratch_types=[
          pltpu.VMEM((nch, ch), jnp.int32),
          pltpu.VMEM((nch, ch), jnp.int32),
      ] + [pltpu.VMEM((ch, D), jnp.float32)] * nbuf
        + [pltpu.VMEM_SHARED((NP, D), jnp.float32)]
        + [pltpu.SemaphoreType.DMA] * (2 * nbuf),
  )
  def prop(g_hbm, src_hbm, dst_hbm, zeros_hbm, out_hbm,
           src_v, dst_v, *rest):
    bufs = rest[:nbuf]
    acc = rest[nbuf]
    sems = rest[nbuf + 1:nbuf + 1 + nbuf]
    ssems = rest[nbuf + 1 + nbuf:]
    cid = lax.axis_index("c")
    sid = lax.axis_index("s")
    wid = sid * NC + cid

    pltpu.sync_copy(src_hbm.at[wid], src_v)
    pltpu.sync_copy(dst_hbm.at[wid], dst_v)
    @pl.when(cid == 0)
    def _():
      @pl.when(sid < NS - 1)
      def _():
        pltpu.sync_copy(g_hbm.at[pl.ds(sid * RPT, RPT)],
                        acc.at[pl.ds(sid * RPT, RPT)])

      @pl.when(sid == NS - 1)
      def _():
        pltpu.sync_copy(g_hbm.at[pl.ds(N - RPT2, RPT2)],
                        acc.at[pl.ds(N - RPT2, RPT2)])
        pltpu.sync_copy(zeros_hbm.at[pl.ds(0, NP - N)],
                        acc.at[pl.ds(N, NP - N)])

    @pl.when(cid == 1)
    def _():
      pltpu.sync_copy(zeros_hbm, acc.at[pl.ds(sid * RPT, RPT)])
    plsc.subcore_barrier()

    def gather(c, b):
      if c >= nbuf:
        pltpu.make_async_copy(bufs[b], acc.at[dst_v.at[c - nbuf]],
                              ssems[b]).wait()
      return pltpu.async_copy(g_hbm.at[src_v.at[c]], bufs[b], sems[b])

    def gather_dyn(c, b):
      @pl.when(c >= nbuf)
      def _():
        pltpu.make_async_copy(bufs[b], acc.at[dst_v.at[c - nbuf]],
                              ssems[b]).wait()
      return pltpu.async_copy(g_hbm.at[src_v.at[c]], bufs[b], sems[b])

    def drain(c, b):
      pltpu.make_async_copy(g_hbm.at[src_v.at[c]], bufs[b], sems[b]).wait()
      pltpu.async_copy(bufs[b], acc.at[dst_v.at[c]], ssems[b], add=True)

    for b in range(nbuf - 1):
      gather(b, b)

    def body(k, c):
      i = nbuf * k
      for j in range(nbuf):
        nxt = i + j + nbuf - 1

        @pl.when(nxt < nch)
        def _():
          gather_dyn(nxt, (j + nbuf - 1) % nbuf)

        drain(i + j, j)
      return c

    lax.fori_loop(0, nch // nbuf, body, 0)
    base = nch - nch % nbuf
    for j in range(nch % nbuf):
      drain(base + j, (base + j) % nbuf)
    for c in range(nch - nbuf, nch):
      pltpu.make_async_copy(bufs[c % nbuf], acc.at[dst_v.at[c]],
                            ssems[c % nbuf]).wait()

    plsc.subcore_barrier()
    pltpu.sync_copy(acc.at[pl.ds(sid * RPT, RPT)],
                    out_hbm.at[cid, pl.ds(sid * RPT, RPT)])

  return prop


_prop128 = _make_prop(128, CH2, NCH2, 5)
_prop64 = _make_prop(64, CH2, NCH2, 8)


@functools.partial(
    pl.kernel,
    out_type=jax.ShapeDtypeStruct((NC, NP, DEG_W), jnp.float32),
    mesh=_mesh,
    compiler_params=pltpu.CompilerParams(use_tc_tiling_on_sc=False),
    scratch_types=[
        pltpu.VMEM((NCH, CH), jnp.int32),
        pltpu.VMEM((CH, DEG_W), jnp.float32),
        pltpu.VMEM_SHARED((NP, DEG_W), jnp.float32),
        pltpu.SemaphoreType.DMA,
    ],
)
def _deg_kernel(ones_hbm, dst_hbm, zeros_hbm, out_hbm, dst_v, ones_v, acc, sem):
  cid = lax.axis_index("c")
  sid = lax.axis_index("s")
  wid = sid * NC + cid

  pltpu.sync_copy(dst_hbm.at[wid], dst_v)
  pltpu.sync_copy(ones_hbm, ones_v)
  pltpu.sync_copy(zeros_hbm, acc.at[pl.ds(sid * RPT, RPT)])
  plsc.subcore_barrier()

  pltpu.async_copy(ones_v, acc.at[dst_v.at[0]], sem, add=True)

  def body(i, c):
    pltpu.async_copy(ones_v, acc.at[dst_v.at[i]], sem, add=True)
    pltpu.make_async_copy(ones_v, acc.at[dst_v.at[i - 1]], sem).wait()
    return c

  lax.fori_loop(1, NCH, body, 0)
  pltpu.make_async_copy(ones_v, acc.at[dst_v.at[NCH - 1]], sem).wait()

  plsc.subcore_barrier()
  pltpu.sync_copy(acc.at[pl.ds(sid * RPT, RPT)],
                  out_hbm.at[cid, pl.ds(sid * RPT, RPT)])


_RB = 1000
_NG = N // _RB


def _row_spec(d):
  return pl.BlockSpec((_RB, d), lambda i: (i, 0))


def _part_spec(d):
  return pl.BlockSpec((NC, _RB, d), lambda i: (0, i, 0))


def _full_spec(r, c):
  return pl.BlockSpec((r, c), lambda i: (0, 0))


def _dis_spec():
  return pl.BlockSpec((_RB, 1), lambda i: (i, 0))


def _tc1_body(degp_ref, x_ref, w1_ref, dis_ref, g1_ref):
  deg = degp_ref[0, :, 0:1] + degp_ref[1, :, 0:1] + 1.0
  dis = lax.rsqrt(deg)
  dis_ref[...] = dis
  xw = jnp.dot(x_ref[...], w1_ref[...], preferred_element_type=jnp.float32)
  g1_ref[...] = dis * xw


def _tc2_body(s1p_ref, dis_ref, b1_ref, w2_ref, g2_ref):
  dis = dis_ref[...]
  s = s1p_ref[0] + s1p_ref[1]
  h1 = jnp.maximum(dis * s + b1_ref[...], 0.0)
  g2_ref[...] = dis * jnp.dot(h1, w2_ref[...],
                              preferred_element_type=jnp.float32)


def _tc3_body(s2p_ref, dis_ref, b2_ref, g3_ref):
  dis = dis_ref[...]
  s = s2p_ref[0] + s2p_ref[1]
  g3_ref[...] = dis * jnp.maximum(dis * s + b2_ref[...], 0.0)


def _tc4_body(s3p_ref, dis_ref, w3_ref, b3_ref, out_ref):
  dis = dis_ref[...]
  p3 = dis * (s3p_ref[0] + s3p_ref[1])
  out_ref[...] = jnp.dot(p3, w3_ref[...],
                         preferred_element_type=jnp.float32) + b3_ref[...]


def kernel(x, edge_index, W1, b1, W2, b2, W3, b3):
  ei = edge_index.astype(jnp.int32)
  src3 = ei[0].reshape(NW, NCH2, CH2)
  dst3 = ei[1].reshape(NW, NCH2, CH2)
  dst3d = ei[1].reshape(NW, NCH, CH)

  ones_deg = jnp.ones((CH, DEG_W), jnp.float32)
  zeros_deg = jnp.zeros((RPT, DEG_W), jnp.float32)
  zeros128 = jnp.zeros((RPT, 128), jnp.float32)
  zeros64 = jnp.zeros((RPT, 64), jnp.float32)

  degp = _deg_kernel(ones_deg, dst3d, zeros_deg)

  dis, g1 = pl.pallas_call(
      _tc1_body,
      grid=(_NG,),
      in_specs=[_part_spec(DEG_W), _row_spec(128), _full_spec(128, 128)],
      out_specs=[_dis_spec(), _row_spec(128)],
      out_shape=[jax.ShapeDtypeStruct((N, 1), jnp.float32),
                 jax.ShapeDtypeStruct((N, 128), jnp.float32)],
  )(degp, x, W1)

  s1p = _prop128(g1, src3, dst3, zeros128)

  g2 = pl.pallas_call(
      _tc2_body,
      grid=(_NG,),
      in_specs=[_part_spec(128), _dis_spec(),
                _full_spec(1, 128), _full_spec(128, 64)],
      out_specs=_row_spec(64),
      out_shape=jax.ShapeDtypeStruct((N, 64), jnp.float32),
  )(s1p, dis, b1.reshape(1, 128), W2)

  s2p = _prop64(g2, src3, dst3, zeros64)

  g3 = pl.pallas_call(
      _tc3_body,
      grid=(_NG,),
      in_specs=[_part_spec(64), _dis_spec(),
                _full_spec(1, 64)],
      out_specs=_row_spec(64),
      out_shape=jax.ShapeDtypeStruct((N, 64), jnp.float32),
  )(s2p, dis, b2.reshape(1, 64))

  s3p = _prop64(g3, src3, dst3, zeros64)

  out = pl.pallas_call(
      _tc4_body,
      grid=(_NG,),
      in_specs=[_part_spec(64), _dis_spec(),
                _full_spec(64, 128), _full_spec(1, 128)],
      out_specs=_row_spec(128),
      out_shape=jax.ShapeDtypeStruct((N, 128), jnp.float32),
  )(s3p, dis, W3, b3.reshape(1, 128))

  return out


# speedup vs baseline: 1.0294x; 1.0294x over previous
import functools

import jax
import jax.numpy as jnp
from jax import lax
from jax.experimental import pallas as pl
from jax.experimental.pallas import tpu as pltpu
from jax.experimental.pallas import tpu_sc as plsc

N = 10000
NP = 10240
E = 320000
NC = 2
NS = 16
NW = NC * NS
EPW = E // NW
CH = 80
NCH = EPW // CH
CH2 = 40
NCH2 = EPW // CH2
CH64 = 128
NCH64 = 79
EP = NW * CH64 * NCH64
EPAD = EP - E
RPT = NP // NS
RPT2 = N - (NS - 1) * RPT
DEG_W = 8

_mesh = plsc.VectorSubcoreMesh(core_axis_name="c", subcore_axis_name="s")


def _make_prop(D, ch, nch, nbuf):
  @functools.partial(
      pl.kernel,
      out_type=jax.ShapeDtypeStruct((NC, NP, D), jnp.float32),
      mesh=_mesh,
      compiler_params=pltpu.CompilerParams(use_tc_tiling_on_sc=False),
      scratch_types=[
          pltpu.VMEM((nch, ch), jnp.int32),
          pltpu.VMEM((nch, ch), jnp.int32),
      ] + [pltpu.VMEM((ch, D), jnp.float32)] * nbuf
        + [pltpu.VMEM_SHARED((NP, D), jnp.float32)]
        + [pltpu.SemaphoreType.DMA] * (2 * nbuf),
  )
  def prop(g_hbm, src_hbm, dst_hbm, zeros_hbm, out_hbm,
           src_v, dst_v, *rest):
    bufs = rest[:nbuf]
    acc = rest[nbuf]
    sems = rest[nbuf + 1:nbuf + 1 + nbuf]
    ssems = rest[nbuf + 1 + nbuf:]
    cid = lax.axis_index("c")
    sid = lax.axis_index("s")
    wid = sid * NC + cid

    pltpu.sync_copy(src_hbm.at[wid], src_v)
    pltpu.sync_copy(dst_hbm.at[wid], dst_v)
    @pl.when(cid == 0)
    def _():
      @pl.when(sid < NS - 1)
      def _():
        pltpu.sync_copy(g_hbm.at[pl.ds(sid * RPT, RPT)],
                        acc.at[pl.ds(sid * RPT, RPT)])

      @pl.when(sid == NS - 1)
      def _():
        pltpu.sync_copy(g_hbm.at[pl.ds(N - RPT2, RPT2)],
                        acc.at[pl.ds(N - RPT2, RPT2)])
        pltpu.sync_copy(zeros_hbm.at[pl.ds(0, NP - N)],
                        acc.at[pl.ds(N, NP - N)])

    @pl.when(cid == 1)
    def _():
      pltpu.sync_copy(zeros_hbm, acc.at[pl.ds(sid * RPT, RPT)])
    plsc.subcore_barrier()

    def gather(c, b):
      if c >= nbuf:
        pltpu.make_async_copy(bufs[b], acc.at[dst_v.at[c - nbuf]],
                              ssems[b]).wait()
      return pltpu.async_copy(g_hbm.at[src_v.at[c]], bufs[b], sems[b])

    def gather_dyn(c, b):
      @pl.when(c >= nbuf)
      def _():
        pltpu.make_async_copy(bufs[b], acc.at[dst_v.at[c - nbuf]],
                              ssems[b]).wait()
      return pltpu.async_copy(g_hbm.at[src_v.at[c]], bufs[b], sems[b])

    def drain(c, b):
      pltpu.make_async_copy(g_hbm.at[src_v.at[c]], bufs[b], sems[b]).wait()
      pltpu.async_copy(bufs[b], acc.at[dst_v.at[c]], ssems[b], add=True)

    for b in range(nbuf - 1):
      gather(b, b)

    def body(k, c):
      i = nbuf * k
      for j in range(nbuf):
        nxt = i + j + nbuf - 1

        @pl.when(nxt < nch)
        def _():
          gather_dyn(nxt, (j + nbuf - 1) % nbuf)

        drain(i + j, j)
      return c

    lax.fori_loop(0, nch // nbuf, body, 0)
    base = nch - nch % nbuf
    for j in range(nch % nbuf):
      drain(base + j, (base + j) % nbuf)
    for c in range(nch - nbuf, nch):
      pltpu.make_async_copy(bufs[c % nbuf], acc.at[dst_v.at[c]],
                            ssems[c % nbuf]).wait()

    plsc.subcore_barrier()
    pltpu.sync_copy(acc.at[pl.ds(sid * RPT, RPT)],
                    out_hbm.at[cid, pl.ds(sid * RPT, RPT)])

  return prop


_prop128 = _make_prop(128, CH2, NCH2, 5)
_prop64 = _make_prop(64, CH2, NCH2, 8)


@functools.partial(
    pl.kernel,
    out_type=jax.ShapeDtypeStruct((NC, NP, DEG_W), jnp.float32),
    mesh=_mesh,
    compiler_params=pltpu.CompilerParams(use_tc_tiling_on_sc=False),
    scratch_types=[
        pltpu.VMEM((NCH, CH), jnp.int32),
        pltpu.VMEM((CH, DEG_W), jnp.float32),
        pltpu.VMEM_SHARED((NP, DEG_W), jnp.float32),
        pltpu.SemaphoreType.DMA,
    ],
)
def _deg_kernel(ones_hbm, dst_hbm, zeros_hbm, out_hbm, dst_v, ones_v, acc, sem):
  cid = lax.axis_index("c")
  sid = lax.axis_index("s")
  wid = sid * NC + cid

  pltpu.sync_copy(dst_hbm.at[wid], dst_v)
  pltpu.sync_copy(ones_hbm, ones_v)
  pltpu.sync_copy(zeros_hbm, acc.at[pl.ds(sid * RPT, RPT)])
  plsc.subcore_barrier()

  pltpu.async_copy(ones_v, acc.at[dst_v.at[0]], sem, add=True)

  def body(i, c):
    pltpu.async_copy(ones_v, acc.at[dst_v.at[i]], sem, add=True)
    pltpu.make_async_copy(ones_v, acc.at[dst_v.at[i - 1]], sem).wait()
    return c

  lax.fori_loop(1, NCH, body, 0)
  pltpu.make_async_copy(ones_v, acc.at[dst_v.at[NCH - 1]], sem).wait()

  plsc.subcore_barrier()
  pltpu.sync_copy(acc.at[pl.ds(sid * RPT, RPT)],
                  out_hbm.at[cid, pl.ds(sid * RPT, RPT)])


_RB = 2000
_NG = N // _RB


def _row_spec(d):
  return pl.BlockSpec((_RB, d), lambda i: (i, 0))


def _part_spec(d):
  return pl.BlockSpec((NC, _RB, d), lambda i: (0, i, 0))


def _full_spec(r, c):
  return pl.BlockSpec((r, c), lambda i: (0, 0))


def _dis_spec():
  return pl.BlockSpec((_RB, 1), lambda i: (i, 0))


def _tc1_body(degp_ref, x_ref, w1_ref, dis_ref, g1_ref):
  deg = degp_ref[0, :, 0:1] + degp_ref[1, :, 0:1] + 1.0
  dis = lax.rsqrt(deg)
  dis_ref[...] = dis
  xw = jnp.dot(x_ref[...], w1_ref[...], preferred_element_type=jnp.float32)
  g1_ref[...] = dis * xw


def _tc2_body(s1p_ref, dis_ref, b1_ref, w2_ref, g2_ref):
  dis = dis_ref[...]
  s = s1p_ref[0] + s1p_ref[1]
  h1 = jnp.maximum(dis * s + b1_ref[...], 0.0)
  g2_ref[...] = dis * jnp.dot(h1, w2_ref[...],
                              preferred_element_type=jnp.float32)


def _tc3_body(s2p_ref, dis_ref, b2_ref, g3_ref):
  dis = dis_ref[...]
  s = s2p_ref[0] + s2p_ref[1]
  g3_ref[...] = dis * jnp.maximum(dis * s + b2_ref[...], 0.0)


def _tc4_body(s3p_ref, dis_ref, w3_ref, b3_ref, out_ref):
  dis = dis_ref[...]
  p3 = dis * (s3p_ref[0] + s3p_ref[1])
  out_ref[...] = jnp.dot(p3, w3_ref[...],
                         preferred_element_type=jnp.float32) + b3_ref[...]


def kernel(x, edge_index, W1, b1, W2, b2, W3, b3):
  ei = edge_index.astype(jnp.int32)
  src3 = ei[0].reshape(NW, NCH2, CH2)
  dst3 = ei[1].reshape(NW, NCH2, CH2)
  dst3d = ei[1].reshape(NW, NCH, CH)

  ones_deg = jnp.ones((CH, DEG_W), jnp.float32)
  zeros_deg = jnp.zeros((RPT, DEG_W), jnp.float32)
  zeros128 = jnp.zeros((RPT, 128), jnp.float32)
  zeros64 = jnp.zeros((RPT, 64), jnp.float32)

  degp = _deg_kernel(ones_deg, dst3d, zeros_deg)

  dis, g1 = pl.pallas_call(
      _tc1_body,
      grid=(_NG,),
      in_specs=[_part_spec(DEG_W), _row_spec(128), _full_spec(128, 128)],
      out_specs=[_dis_spec(), _row_spec(128)],
      out_shape=[jax.ShapeDtypeStruct((N, 1), jnp.float32),
                 jax.ShapeDtypeStruct((N, 128), jnp.float32)],
  )(degp, x, W1)

  s1p = _prop128(g1, src3, dst3, zeros128)

  g2 = pl.pallas_call(
      _tc2_body,
      grid=(_NG,),
      in_specs=[_part_spec(128), _dis_spec(),
                _full_spec(1, 128), _full_spec(128, 64)],
      out_specs=_row_spec(64),
      out_shape=jax.ShapeDtypeStruct((N, 64), jnp.float32),
  )(s1p, dis, b1.reshape(1, 128), W2)

  s2p = _prop64(g2, src3, dst3, zeros64)

  g3 = pl.pallas_call(
      _tc3_body,
      grid=(_NG,),
      in_specs=[_part_spec(64), _dis_spec(),
                _full_spec(1, 64)],
      out_specs=_row_spec(64),
      out_shape=jax.ShapeDtypeStruct((N, 64), jnp.float32),
  )(s2p, dis, b2.reshape(1, 64))

  s3p = _prop64(g3, src3, dst3, zeros64)

  out = pl.pallas_call(
      _tc4_body,
      grid=(_NG,),
      in_specs=[_part_spec(64), _dis_spec(),
                _full_spec(64, 128), _full_spec(1, 128)],
      out_specs=_row_spec(128),
      out_shape=jax.ShapeDtypeStruct((N, 128), jnp.float32),
  )(s3p, dis, W3, b3.reshape(1, 128))

  return out
